# Initial kernel scaffold; baseline (speedup 1.0000x reference)
import jax
import jax.numpy as jnp
from jax.experimental import pallas as pl


def kernel(input, weight):
    raise NotImplementedError("write your pallas kernel here")


# speedup vs baseline: 9.2740x; 9.2740x over previous
import functools

import jax
import jax.numpy as jnp
from jax import lax
from jax.experimental import pallas as pl
from jax.experimental.pallas import tpu as pltpu
from jax.experimental.pallas import tpu_sc as plsc

CH = 128


@functools.lru_cache(maxsize=None)
def _make_gather(V, D, NROWS):
    info = plsc.get_sparse_core_info()
    NC, NS = info.num_cores, info.num_subcores
    NW = NC * NS
    assert NROWS % (CH * NW) == 0
    nch = NROWS // (CH * NW)
    mesh = plsc.VectorSubcoreMesh(core_axis_name="c", subcore_axis_name="s")

    @functools.partial(
        pl.kernel,
        mesh=mesh,
        out_type=jax.ShapeDtypeStruct((NROWS, D), jnp.float32),
        scratch_types=[
            pltpu.VMEM((nch, CH), jnp.int32),
            pltpu.VMEM((2, CH, D), jnp.float32),
            pltpu.SemaphoreType.DMA,
            pltpu.SemaphoreType.DMA,
        ],
    )
    def k(table_hbm, idx_hbm, out_hbm, idx_v, rows_v, gsem, psem):
        wid = lax.axis_index("s") * NC + lax.axis_index("c")
        pltpu.sync_copy(idx_hbm.at[pl.ds(wid * nch, nch)], idx_v)
        row_base = wid * nch * CH

        def g_start(t, b):
            pltpu.async_copy(table_hbm.at[idx_v.at[t]], rows_v.at[b], gsem)

        def g_wait(b):
            pltpu.make_async_copy(
                table_hbm.at[idx_v.at[0]], rows_v.at[b], gsem
            ).wait()

        def p_start(t, b):
            pltpu.async_copy(
                rows_v.at[b], out_hbm.at[pl.ds(row_base + t * CH, CH)], psem
            )

        def p_wait(b):
            pltpu.make_async_copy(
                rows_v.at[b], out_hbm.at[pl.ds(row_base, CH)], psem
            ).wait()

        g_start(0, 0)
        g_start(1, 1)

        def body(i, _):
            t = i * 2
            for b in range(2):
                tt = t + b
                g_wait(b)
                p_start(tt, b)
                p_wait(b)

                @pl.when(tt + 2 < nch)
                def _():
                    g_start(tt + 2, b)

            return ()

        lax.fori_loop(0, nch // 2, body, (), unroll=False)

    return k


def kernel(input, weight):
    B, S = input.shape
    V, D = weight.shape
    idx = input.reshape(-1).astype(jnp.int32).reshape(-1, CH)
    out = _make_gather(V, D, B * S)(weight, idx)
    return out.reshape(B, S, D)


# speedup vs baseline: 9.3355x; 1.0066x over previous
import functools

import jax
import jax.numpy as jnp
from jax import lax
from jax.experimental import pallas as pl
from jax.experimental.pallas import tpu as pltpu
from jax.experimental.pallas import tpu_sc as plsc

CH = 128
NBUF = 5
G = 3


@functools.lru_cache(maxsize=None)
def _make_gather(V, D, NROWS):
    info = plsc.get_sparse_core_info()
    NC, NS = info.num_cores, info.num_subcores
    NW = NC * NS
    assert NROWS % (CH * NW) == 0
    nch = NROWS // (CH * NW)
    mesh = plsc.VectorSubcoreMesh(core_axis_name="c", subcore_axis_name="s")

    @functools.partial(
        pl.kernel,
        mesh=mesh,
        out_type=jax.ShapeDtypeStruct((NROWS, D), jnp.float32),
        scratch_types=[
            pltpu.VMEM((nch, CH), jnp.int32),
            pltpu.VMEM((NBUF, CH, D), jnp.float32),
            pltpu.SemaphoreType.DMA,
            pltpu.SemaphoreType.DMA,
        ],
    )
    def k(table_hbm, idx_hbm, out_hbm, idx_v, rows_v, gsem, psem):
        wid = lax.axis_index("s") * NC + lax.axis_index("c")
        pltpu.sync_copy(idx_hbm.at[pl.ds(wid * nch, nch)], idx_v)
        row_base = wid * nch * CH

        def g_start(t, b):
            pltpu.async_copy(table_hbm.at[idx_v.at[t]], rows_v.at[b], gsem)

        def g_wait():
            pltpu.make_async_copy(
                table_hbm.at[idx_v.at[0]], rows_v.at[0], gsem
            ).wait()

        def p_start(t, b):
            pltpu.async_copy(
                rows_v.at[b], out_hbm.at[pl.ds(row_base + t * CH, CH)], psem
            )

        def p_wait():
            pltpu.make_async_copy(
                rows_v.at[0], out_hbm.at[pl.ds(row_base, CH)], psem
            ).wait()

        for t in range(G):
            g_start(t, t)

        for tt in range(NBUF - G):
            g_start(tt + G, tt + G)
            g_wait()
            p_start(tt, tt)

        def body(tt, _):
            p_wait()
            g_start(tt + G, lax.rem(tt + G, NBUF))
            g_wait()
            p_start(tt, lax.rem(tt, NBUF))
            return ()

        lax.fori_loop(NBUF - G, nch - G, body, (), unroll=False)

        for tt in range(nch - G, nch):
            g_wait()
            p_start(tt, tt % NBUF)
        for _ in range(NBUF):
            p_wait()

    return k


def kernel(input, weight):
    B, S = input.shape
    V, D = weight.shape
    idx = input.reshape(-1).astype(jnp.int32).reshape(-1, CH)
    out = _make_gather(V, D, B * S)(weight, idx)
    return out.reshape(B, S, D)


# speedup vs baseline: 9.6366x; 1.0323x over previous
import functools

import jax
import jax.numpy as jnp
from jax import lax
from jax.experimental import pallas as pl
from jax.experimental.pallas import tpu as pltpu
from jax.experimental.pallas import tpu_sc as plsc

CH = 128
NBUF = 3
NSP = 2


@functools.lru_cache(maxsize=None)
def _make_gather(V, D, NROWS):
    info = plsc.get_sparse_core_info()
    NC, NS = info.num_cores, info.num_subcores
    NW = NC * NS
    assert NROWS % (CH * NW) == 0
    nch = NROWS // (CH * NW)
    mesh = plsc.VectorSubcoreMesh(core_axis_name="c", subcore_axis_name="s")

    @functools.partial(
        pl.kernel,
        mesh=mesh,
        out_type=jax.ShapeDtypeStruct((NROWS, D), jnp.float32),
        scratch_types=[
            pltpu.VMEM((nch, CH), jnp.int32),
            pltpu.VMEM((NBUF, CH, D), jnp.float32),
            pltpu.VMEM_SHARED((NS, NSP, CH, D), jnp.float32),
            pltpu.SemaphoreType.DMA,
            pltpu.SemaphoreType.DMA,
            pltpu.SemaphoreType.DMA,
        ],
    )
    def k(table_hbm, idx_hbm, out_hbm, idx_v, rows_v, spm, gsem, s1, s2):
        wid = lax.axis_index("s") * NC + lax.axis_index("c")
        sid = lax.axis_index("s")
        pltpu.sync_copy(idx_hbm.at[pl.ds(wid * nch, nch)], idx_v)
        row_base = wid * nch * CH

        def g_start(t, b):
            pltpu.async_copy(table_hbm.at[idx_v.at[t]], rows_v.at[b], gsem)

        def g_wait():
            pltpu.make_async_copy(
                table_hbm.at[idx_v.at[0]], rows_v.at[0], gsem
            ).wait()

        def l1_start(t, b):
            pltpu.async_copy(rows_v.at[b], spm.at[sid, lax.rem(t, NSP)], s1)

        def l1_wait():
            pltpu.make_async_copy(rows_v.at[0], spm.at[sid, 0], s1).wait()

        def l2_start(t):
            pltpu.async_copy(
                spm.at[sid, lax.rem(t, NSP)],
                out_hbm.at[pl.ds(row_base + t * CH, CH)],
                s2,
            )

        def l2_wait():
            pltpu.make_async_copy(
                spm.at[sid, 0], out_hbm.at[pl.ds(row_base, CH)], s2
            ).wait()

        g_start(0, 0)
        g_start(1, 1)
        g_wait()
        l1_start(0, 0)
        g_start(2, 2)
        g_wait()
        l1_start(1, 1)
        l1_wait()
        l2_start(0)
        g_start(3, 0)

        def body(tt, _):
            l2_wait()
            g_wait()
            l1_start(tt, lax.rem(tt, NBUF))
            l1_wait()
            l2_start(tt - 1)
            g_start(tt + 2, lax.rem(tt + 2, NBUF))
            return ()

        lax.fori_loop(2, nch - 2, body, (), unroll=False)

        for tt in range(nch - 2, nch):
            l2_wait()
            g_wait()
            l1_start(tt, tt % NBUF)
            l1_wait()
            l2_start(tt - 1)
        l1_wait()
        l2_start(nch - 1)
        l2_wait()
        l2_wait()

    return k


def kernel(input, weight):
    B, S = input.shape
    V, D = weight.shape
    idx = input.reshape(-1).astype(jnp.int32).reshape(-1, CH)
    out = _make_gather(V, D, B * S)(weight, idx)
    return out.reshape(B, S, D)


# speedup vs baseline: 9.6403x; 1.0004x over previous
import functools

import jax
import jax.numpy as jnp
from jax import lax
from jax.experimental import pallas as pl
from jax.experimental.pallas import tpu as pltpu
from jax.experimental.pallas import tpu_sc as plsc

CH = 128
NBUF = 3
NSP = 3


@functools.lru_cache(maxsize=None)
def _make_gather(V, D, NROWS):
    info = plsc.get_sparse_core_info()
    NC, NS = info.num_cores, info.num_subcores
    NW = NC * NS
    assert NROWS % (CH * NW) == 0
    nch = NROWS // (CH * NW)
    mesh = plsc.VectorSubcoreMesh(core_axis_name="c", subcore_axis_name="s")

    @functools.partial(
        pl.kernel,
        mesh=mesh,
        out_type=jax.ShapeDtypeStruct((NROWS, D), jnp.float32),
        scratch_types=[
            pltpu.VMEM((nch, CH), jnp.int32),
            pltpu.VMEM((NBUF, CH, D), jnp.float32),
            pltpu.VMEM_SHARED((NS, NSP, CH, D), jnp.float32),
            pltpu.SemaphoreType.DMA,
            pltpu.SemaphoreType.DMA,
            pltpu.SemaphoreType.DMA,
        ],
    )
    def k(table_hbm, idx_hbm, out_hbm, idx_v, rows_v, spm, gsem, s1, s2):
        wid = lax.axis_index("s") * NC + lax.axis_index("c")
        sid = lax.axis_index("s")
        pltpu.sync_copy(idx_hbm.at[pl.ds(wid * nch, nch)], idx_v)
        row_base = wid * nch * CH

        def g_start(t, b):
            pltpu.async_copy(table_hbm.at[idx_v.at[t]], rows_v.at[b], gsem)

        def g_wait():
            pltpu.make_async_copy(
                table_hbm.at[idx_v.at[0]], rows_v.at[0], gsem
            ).wait()

        def l1_start(t, b):
            pltpu.async_copy(rows_v.at[b], spm.at[sid, lax.rem(t, NSP)], s1)

        def l1_wait():
            pltpu.make_async_copy(rows_v.at[0], spm.at[sid, 0], s1).wait()

        def l2_start(t):
            pltpu.async_copy(
                spm.at[sid, lax.rem(t, NSP)],
                out_hbm.at[pl.ds(row_base + t * CH, CH)],
                s2,
            )

        def l2_wait():
            pltpu.make_async_copy(
                spm.at[sid, 0], out_hbm.at[pl.ds(row_base, CH)], s2
            ).wait()

        g_start(0, 0)
        g_start(1, 1)
        g_wait()
        l1_start(0, 0)
        g_start(2, 2)
        g_wait()
        l1_start(1, 1)
        l1_wait()
        l2_start(0)
        g_start(3, 0)
        g_wait()
        l1_start(2, 2)
        l1_wait()
        l2_start(1)
        g_start(4, 1)

        def body(tt, _):
            l2_wait()
            g_wait()
            l1_start(tt, lax.rem(tt, NBUF))
            l1_wait()
            l2_start(tt - 1)
            g_start(tt + 2, lax.rem(tt + 2, NBUF))
            return ()

        lax.fori_loop(3, nch - 2, body, (), unroll=False)

        for tt in range(nch - 2, nch):
            l2_wait()
            g_wait()
            l1_start(tt, tt % NBUF)
            l1_wait()
            l2_start(tt - 1)
        l1_wait()
        l2_start(nch - 1)
        l2_wait()
        l2_wait()
        l2_wait()

    return k


def kernel(input, weight):
    B, S = input.shape
    V, D = weight.shape
    idx = input.reshape(-1).astype(jnp.int32).reshape(-1, CH)
    out = _make_gather(V, D, B * S)(weight, idx)
    return out.reshape(B, S, D)
